# baseline (speedup 1.0000x reference)
import functools

import jax
import jax.numpy as jnp
from jax import lax
from jax.experimental import pallas as pl
from jax.experimental.pallas import tpu as pltpu
from jax.experimental.pallas import tpu_sc as plsc

VOCAB = 1_000_000
HIDDEN = 32
BATCH = 4096
HIST = 50
NC, NS, LANES = 2, 16, 16
NW = NC * NS
BPW = BATCH // NW


def _sc_body(ctxT_hbm, t_hbm, in_emb_hbm, out_emb_hbm, con_hbm, tgt_hbm,
             ctx_v, tidx_v, rows0, rows1, acc, tgt_v, sem0, sem1, semt):
    wid = lax.axis_index("s") * NC + lax.axis_index("c")
    base = wid * BPW

    pltpu.sync_copy(ctxT_hbm.at[:, pl.ds(base, BPW)], ctx_v)
    pltpu.sync_copy(t_hbm.at[pl.ds(base, BPW)], tidx_v)

    pltpu.async_copy(out_emb_hbm.at[tidx_v], tgt_v, semt)

    zeros = jnp.zeros((LANES,), jnp.float32)

    @pl.loop(0, BPW)
    def _zero(i):
        acc[i, pl.ds(0, LANES)] = zeros
        acc[i, pl.ds(LANES, LANES)] = zeros

    pltpu.async_copy(in_emb_hbm.at[ctx_v.at[0]], rows0, sem0)
    pltpu.async_copy(in_emb_hbm.at[ctx_v.at[1]], rows1, sem1)

    @pl.loop(0, HIST, step=2)
    def _h(h):
        for b, (rows, sem) in enumerate(((rows0, sem0), (rows1, sem1))):
            hc = h + b
            pltpu.make_async_copy(in_emb_hbm.at[ctx_v.at[hc]], rows, sem).wait()

            @pl.loop(0, BPW, unroll=4)
            def _acc(i):
                plsc.addupdate(acc.at[i, pl.ds(0, LANES)],
                               rows[i, pl.ds(0, LANES)])
                plsc.addupdate(acc.at[i, pl.ds(LANES, LANES)],
                               rows[i, pl.ds(LANES, LANES)])

            @pl.when(hc + 2 < HIST)
            def _next():
                pltpu.async_copy(in_emb_hbm.at[ctx_v.at[hc + 2]], rows, sem)

    pltpu.sync_copy(acc, con_hbm.at[pl.ds(base, BPW)])
    pltpu.make_async_copy(out_emb_hbm.at[tidx_v], tgt_v, semt).wait()
    pltpu.sync_copy(tgt_v, tgt_hbm.at[pl.ds(base, BPW)])


def _sc_gather(ctxT, t_flat, in_emb, out_emb):
    mesh = plsc.VectorSubcoreMesh(core_axis_name="c", subcore_axis_name="s",
                                  num_cores=NC, num_subcores=NS)
    f = pl.kernel(
        _sc_body,
        out_type=(jax.ShapeDtypeStruct((BATCH, HIDDEN), jnp.float32),
                  jax.ShapeDtypeStruct((BATCH, HIDDEN), jnp.float32)),
        mesh=mesh,
        compiler_params=pltpu.CompilerParams(use_tc_tiling_on_sc=False),
        scratch_types=[
            pltpu.VMEM((HIST, BPW), jnp.int32),
            pltpu.VMEM((BPW,), jnp.int32),
            pltpu.VMEM((BPW, HIDDEN), jnp.float32),
            pltpu.VMEM((BPW, HIDDEN), jnp.float32),
            pltpu.VMEM((BPW, HIDDEN), jnp.float32),
            pltpu.VMEM((BPW, HIDDEN), jnp.float32),
            pltpu.SemaphoreType.DMA,
            pltpu.SemaphoreType.DMA,
            pltpu.SemaphoreType.DMA,
        ],
    )
    return f(ctxT, t_flat, in_emb, out_emb)


def _mm_body(a_ref, b_ref, o_ref):
    o_ref[...] = lax.dot_general(a_ref[...], b_ref[...],
                                 (((1,), (1,)), ((), ())),
                                 preferred_element_type=jnp.float32)


def _tc_matmul(con, tgt):
    blk = 1024
    return pl.pallas_call(
        _mm_body,
        grid=(BATCH // blk, BATCH // blk),
        in_specs=[pl.BlockSpec((blk, HIDDEN), lambda i, j: (i, 0)),
                  pl.BlockSpec((blk, HIDDEN), lambda i, j: (j, 0))],
        out_specs=pl.BlockSpec((blk, blk), lambda i, j: (i, j)),
        out_shape=jax.ShapeDtypeStruct((BATCH, BATCH), jnp.float32),
    )(con, tgt)


def kernel(contexts, t, in_emb, out_emb):
    ctxT = contexts.T
    t_flat = t.reshape(BATCH)
    con, tgt = _sc_gather(ctxT, t_flat, in_emb, out_emb)
    return _tc_matmul(con, tgt)
